# baseline (speedup 1.0000x reference)
import functools

import jax
import jax.numpy as jnp
import numpy as np
from jax.experimental import pallas as pl
from jax.experimental.pallas import tpu as pltpu

_B = 4096
_K = 30
_FPF = 80
_CTRL = 32
_HID = 2048
_TRUNK = 1024
_IPP = 8
_EMB = [('action', 400, 32), ('jumps', 8, 4), ('character', 34, 12),
        ('l_cancel', 4, 2), ('hurtbox', 4, 2), ('ground', 128, 8),
        ('last_attack', 64, 8), ('state_age', 128, 8)]
_STAGE = ('stage', 32, 8)
_HEADS = [('continuous', 8), ('binary', 6), ('velocity', 10), ('dynamics', 16),
          ('p0_action', 400), ('p1_action', 400), ('p0_jumps', 8), ('p1_jumps', 8),
          ('p0_l_cancel', 4), ('p1_l_cancel', 4), ('p0_hurtbox', 4), ('p1_hurtbox', 4),
          ('p0_ground', 128), ('p1_ground', 128), ('p0_last_attack', 64),
          ('p1_last_attack', 64)]
_EMB_PER_FRAME = 2 * sum(d for _, _, d in _EMB) + _STAGE[2]
_FRAME = _FPF + _EMB_PER_FRAME
_NF = _K * _FPF
_NE = _K * _EMB_PER_FRAME
_IN = _FRAME * _K + _CTRL
_HOUT = sum(o for _, o in _HEADS)
_HPAD = 1280

_BT = 256


def _w1_perm():
    perm_f = np.empty(_NF, np.int32)
    perm_e = np.empty(_NE, np.int32)
    for k in range(_K):
        perm_f[k * _FPF:(k + 1) * _FPF] = np.arange(k * _FRAME, k * _FRAME + _FPF)
        perm_e[k * _EMB_PER_FRAME:(k + 1) * _EMB_PER_FRAME] = np.arange(
            k * _FRAME + _FPF, (k + 1) * _FRAME)
    return np.concatenate([perm_f, perm_e])


_PERM = _w1_perm()


def _mlp_body(xf_ref, xe_ref, xc_ref, w1f_ref, w1e_ref, w1c_ref, b1_ref,
              w2_ref, b2_ref, wh_ref, bh_ref, out_ref):
    f32 = jnp.float32
    h = jnp.dot(xf_ref[...], w1f_ref[...], preferred_element_type=f32)
    h += jnp.dot(xe_ref[...], w1e_ref[...], preferred_element_type=f32)
    h += jnp.dot(xc_ref[...], w1c_ref[...], preferred_element_type=f32)
    h = jnp.maximum(h + b1_ref[...], 0.0).astype(jnp.bfloat16)
    h2 = jnp.dot(h, w2_ref[...], preferred_element_type=f32)
    h2 = jnp.maximum(h2 + b2_ref[...], 0.0).astype(jnp.bfloat16)
    out_ref[...] = jnp.dot(h2, wh_ref[...], preferred_element_type=f32) + bh_ref[...]


def _mlp(xf, xe, xc, w1f, w1e, w1c, b1, w2, b2, wh, bh):
    const = lambda i: (0, 0)
    return pl.pallas_call(
        _mlp_body,
        grid=(_B // _BT,),
        in_specs=[
            pl.BlockSpec((_BT, _NF), lambda i: (i, 0)),
            pl.BlockSpec((_BT, _NE), lambda i: (i, 0)),
            pl.BlockSpec((_BT, _CTRL), lambda i: (i, 0)),
            pl.BlockSpec((_NF, _HID), const),
            pl.BlockSpec((_NE, _HID), const),
            pl.BlockSpec((_CTRL, _HID), const),
            pl.BlockSpec((1, _HID), const),
            pl.BlockSpec((_HID, _TRUNK), const),
            pl.BlockSpec((1, _TRUNK), const),
            pl.BlockSpec((_TRUNK, _HPAD), const),
            pl.BlockSpec((1, _HPAD), const),
        ],
        out_specs=pl.BlockSpec((_BT, _HPAD), lambda i: (i, 0)),
        out_shape=jax.ShapeDtypeStruct((_B, _HPAD), jnp.float32),
        compiler_params=pltpu.CompilerParams(
            dimension_semantics=("arbitrary",),
        ),
    )(xf, xe, xc, w1f, w1e, w1c, b1, w2, b2, wh, bh)


def _gather_emb(int_ctx, params):
    parts = []
    for base in (0, _IPP):
        for j, (name, _, _) in enumerate(_EMB):
            parts.append(params[name + '_embed'].astype(jnp.bfloat16)[int_ctx[:, :, base + j]])
    parts.append(params[_STAGE[0] + '_embed'].astype(jnp.bfloat16)[int_ctx[:, :, 2 * _IPP]])
    order = parts[:len(_EMB)] + parts[len(_EMB):2 * len(_EMB)] + parts[-1:]
    fe = jnp.concatenate(order, axis=-1)
    return fe.reshape(_B, _NE)


def kernel(float_ctx, int_ctx, next_ctrl, params):
    bf16 = jnp.bfloat16
    xf = float_ctx.reshape(_B, _NF).astype(bf16)
    xc = next_ctrl.astype(bf16)
    xe = _gather_emb(int_ctx, params)

    w1p = params['W1'][_PERM[:_NF + _NE]]
    w1f = w1p[:_NF].astype(bf16)
    w1e = w1p[_NF:].astype(bf16)
    w1c = params['W1'][_FRAME * _K:].astype(bf16)
    b1 = params['b1'].reshape(1, _HID)
    w2 = params['W2'].astype(bf16)
    b2 = params['b2'].reshape(1, _TRUNK)
    wh = jnp.concatenate([params[n + '_W'] for n, _ in _HEADS], axis=1)
    wh = jnp.pad(wh, ((0, 0), (0, _HPAD - _HOUT))).astype(bf16)
    bh = jnp.pad(jnp.concatenate([params[n + '_b'] for n, _ in _HEADS]),
                 (0, _HPAD - _HOUT)).reshape(1, _HPAD)

    out = _mlp(xf, xe, xc, w1f, w1e, w1c, b1, w2, b2, wh, bh)

    res, off = {}, 0
    for name, o in _HEADS:
        res[name] = out[:, off:off + o]
        off += o
    return res


# speedup vs baseline: 7.4828x; 7.4828x over previous
import functools

import jax
import jax.numpy as jnp
import numpy as np
from jax import lax
from jax.experimental import pallas as pl
from jax.experimental.pallas import tpu as pltpu
from jax.experimental.pallas import tpu_sc as plsc

_B = 4096
_K = 30
_FPF = 80
_CTRL = 32
_HID = 2048
_TRUNK = 1024
_IPP = 8
_EMB = [('action', 400, 32), ('jumps', 8, 4), ('character', 34, 12),
        ('l_cancel', 4, 2), ('hurtbox', 4, 2), ('ground', 128, 8),
        ('last_attack', 64, 8), ('state_age', 128, 8)]
_STAGE = ('stage', 32, 8)
_HEADS = [('continuous', 8), ('binary', 6), ('velocity', 10), ('dynamics', 16),
          ('p0_action', 400), ('p1_action', 400), ('p0_jumps', 8), ('p1_jumps', 8),
          ('p0_l_cancel', 4), ('p1_l_cancel', 4), ('p0_hurtbox', 4), ('p1_hurtbox', 4),
          ('p0_ground', 128), ('p1_ground', 128), ('p0_last_attack', 64),
          ('p1_last_attack', 64)]
_EMB_PER_FRAME = 2 * sum(d for _, _, d in _EMB) + _STAGE[2]
_FRAME = _FPF + _EMB_PER_FRAME
_NF = _K * _FPF
_NE = _K * _EMB_PER_FRAME
_IN = _FRAME * _K + _CTRL
_HOUT = sum(o for _, o in _HEADS)
_HPAD = 1280

_BT = 256


def _w1_perm():
    perm_f = np.empty(_NF, np.int32)
    perm_e = np.empty(_NE, np.int32)
    for k in range(_K):
        perm_f[k * _FPF:(k + 1) * _FPF] = np.arange(k * _FRAME, k * _FRAME + _FPF)
        perm_e[k * _EMB_PER_FRAME:(k + 1) * _EMB_PER_FRAME] = np.arange(
            k * _FRAME + _FPF, (k + 1) * _FRAME)
    return np.concatenate([perm_f, perm_e])


_PERM = _w1_perm()


def _mlp_body(xf_ref, xe_ref, xc_ref, w1f_ref, w1e_ref, w1c_ref, b1_ref,
              w2_ref, b2_ref, wh_ref, bh_ref, out_ref):
    f32 = jnp.float32
    h = jnp.dot(xf_ref[...], w1f_ref[...], preferred_element_type=f32)
    h += jnp.dot(xe_ref[...], w1e_ref[...], preferred_element_type=f32)
    h += jnp.dot(xc_ref[...], w1c_ref[...], preferred_element_type=f32)
    h = jnp.maximum(h + b1_ref[...], 0.0).astype(jnp.bfloat16)
    h2 = jnp.dot(h, w2_ref[...], preferred_element_type=f32)
    h2 = jnp.maximum(h2 + b2_ref[...], 0.0).astype(jnp.bfloat16)
    out_ref[...] = jnp.dot(h2, wh_ref[...], preferred_element_type=f32) + bh_ref[...]


def _mlp(xf, xe, xc, w1f, w1e, w1c, b1, w2, b2, wh, bh):
    const = lambda i: (0, 0)
    return pl.pallas_call(
        _mlp_body,
        grid=(_B // _BT,),
        in_specs=[
            pl.BlockSpec((_BT, _NF), lambda i: (i, 0)),
            pl.BlockSpec((_BT, _NE), lambda i: (i, 0)),
            pl.BlockSpec((_BT, _CTRL), lambda i: (i, 0)),
            pl.BlockSpec((_NF, _HID), const),
            pl.BlockSpec((_NE, _HID), const),
            pl.BlockSpec((_CTRL, _HID), const),
            pl.BlockSpec((1, _HID), const),
            pl.BlockSpec((_HID, _TRUNK), const),
            pl.BlockSpec((1, _TRUNK), const),
            pl.BlockSpec((_TRUNK, _HPAD), const),
            pl.BlockSpec((1, _HPAD), const),
        ],
        out_specs=pl.BlockSpec((_BT, _HPAD), lambda i: (i, 0)),
        out_shape=jax.ShapeDtypeStruct((_B, _HPAD), jnp.float32),
        compiler_params=pltpu.CompilerParams(
            dimension_semantics=("arbitrary",),
        ),
    )(xf, xe, xc, w1f, w1e, w1c, b1, w2, b2, wh, bh)


_SLOT_TABLE = list(range(8)) + list(range(8)) + [8]
_TAB_DIMS = [d for _, _, d in _EMB] + [_STAGE[2]]
_TAB_VOCAB = [v for _, v, _ in _EMB] + [_STAGE[1]]
_TAB_WOFF = np.concatenate([[0], np.cumsum(
    [v * d // 2 for v, d in zip(_TAB_VOCAB, _TAB_DIMS)])]).astype(np.int32)
_TAB_WORDS = int(_TAB_WOFF[-1])
_TAB_WPAD = (_TAB_WORDS + 15) // 16 * 16
_SLOT_DW = [_TAB_DIMS[t] // 2 for t in _SLOT_TABLE]
_SLOT_CWO = np.concatenate([[0], np.cumsum(_SLOT_DW)]).astype(np.int32)
_WPF = int(_SLOT_CWO[-1])
_NSLOT = 2 * _IPP + 1
_NEW = _K * _WPF
_BC = 16
_NW = 32


def _sc_gather_body(tab_hbm, idx_hbm, out_hbm, tab_v, idx_v, out_v):
    wid = lax.axis_index("c") * 16 + lax.axis_index("s")
    pltpu.sync_copy(tab_hbm, tab_v)
    nidx = _NSLOT * _K
    rows_i = lax.iota(jnp.int32, 16) * nidx
    rows_o = lax.iota(jnp.int32, 16) * _NEW
    chunks = _B // _BC // _NW

    def chunk_body(i, carry):
        b0 = (wid * chunks + i) * _BC
        pltpu.sync_copy(idx_hbm.at[pl.ds(b0 * nidx, _BC * nidx)], idx_v)

        def k_body(k, carry2):
            for s in range(_NSLOT):
                idxv = plsc.load_gather(idx_v, [rows_i + (k * _NSLOT + s)])
                wbase = idxv * _SLOT_DW[s] + int(_TAB_WOFF[_SLOT_TABLE[s]])
                obase = rows_o + (k * _WPF + int(_SLOT_CWO[s]))
                for j in range(_SLOT_DW[s]):
                    data = plsc.load_gather(tab_v, [wbase + j])
                    plsc.store_scatter(out_v, [obase + j], data)
            return carry2

        lax.fori_loop(0, _K, k_body, 0)
        pltpu.sync_copy(out_v, out_hbm.at[pl.ds(b0 * _NEW, _BC * _NEW)])
        return carry

    lax.fori_loop(0, chunks, chunk_body, 0)


def _sc_gather(tab_words, idx_flat):
    mesh = plsc.VectorSubcoreMesh(core_axis_name="c", subcore_axis_name="s")
    f = functools.partial(
        pl.kernel,
        mesh=mesh,
        out_type=jax.ShapeDtypeStruct((_B * _NEW,), jnp.int32),
        scratch_types=[
            pltpu.VMEM((_TAB_WPAD,), jnp.int32),
            pltpu.VMEM((_BC * _NSLOT * _K,), jnp.int32),
            pltpu.VMEM((_BC * _NEW,), jnp.int32),
        ],
        compiler_params=pltpu.CompilerParams(needs_layout_passes=False),
    )(_sc_gather_body)
    return f(tab_words, idx_flat)


def _pack_tables(params):
    parts = []
    for name, v, d in _EMB + [_STAGE]:
        e = params[name + '_embed'].astype(jnp.bfloat16)
        parts.append(lax.bitcast_convert_type(
            e.reshape(v, d // 2, 2), jnp.int32).reshape(-1))
    flat = jnp.concatenate(parts)
    return jnp.pad(flat, (0, _TAB_WPAD - _TAB_WORDS))


def _gather_emb_sc(int_ctx, params):
    tab_words = _pack_tables(params)
    idx_flat = int_ctx.reshape(_B * _K * _NSLOT)
    words = _sc_gather(tab_words, idx_flat).reshape(_B, _NEW)
    return lax.bitcast_convert_type(words, jnp.bfloat16).reshape(_B, _NE)


def _gather_emb(int_ctx, params):
    parts = []
    for base in (0, _IPP):
        for j, (name, _, _) in enumerate(_EMB):
            parts.append(params[name + '_embed'].astype(jnp.bfloat16)[int_ctx[:, :, base + j]])
    parts.append(params[_STAGE[0] + '_embed'].astype(jnp.bfloat16)[int_ctx[:, :, 2 * _IPP]])
    order = parts[:len(_EMB)] + parts[len(_EMB):2 * len(_EMB)] + parts[-1:]
    fe = jnp.concatenate(order, axis=-1)
    return fe.reshape(_B, _NE)


def kernel(float_ctx, int_ctx, next_ctrl, params):
    bf16 = jnp.bfloat16
    xf = float_ctx.reshape(_B, _NF).astype(bf16)
    xc = next_ctrl.astype(bf16)
    xe = _gather_emb_sc(int_ctx, params)

    w1p = params['W1'][_PERM[:_NF + _NE]]
    w1f = w1p[:_NF].astype(bf16)
    w1e = w1p[_NF:].astype(bf16)
    w1c = params['W1'][_FRAME * _K:].astype(bf16)
    b1 = params['b1'].reshape(1, _HID)
    w2 = params['W2'].astype(bf16)
    b2 = params['b2'].reshape(1, _TRUNK)
    wh = jnp.concatenate([params[n + '_W'] for n, _ in _HEADS], axis=1)
    wh = jnp.pad(wh, ((0, 0), (0, _HPAD - _HOUT))).astype(bf16)
    bh = jnp.pad(jnp.concatenate([params[n + '_b'] for n, _ in _HEADS]),
                 (0, _HPAD - _HOUT)).reshape(1, _HPAD)

    out = _mlp(xf, xe, xc, w1f, w1e, w1c, b1, w2, b2, wh, bh)

    res, off = {}, 0
    for name, o in _HEADS:
        res[name] = out[:, off:off + o]
        off += o
    return res


# speedup vs baseline: 8.4440x; 1.1285x over previous
import functools

import jax
import jax.numpy as jnp
import numpy as np
from jax import lax
from jax.experimental import pallas as pl
from jax.experimental.pallas import tpu as pltpu
from jax.experimental.pallas import tpu_sc as plsc

_B = 4096
_K = 30
_FPF = 80
_CTRL = 32
_HID = 2048
_TRUNK = 1024
_IPP = 8
_EMB = [('action', 400, 32), ('jumps', 8, 4), ('character', 34, 12),
        ('l_cancel', 4, 2), ('hurtbox', 4, 2), ('ground', 128, 8),
        ('last_attack', 64, 8), ('state_age', 128, 8)]
_STAGE = ('stage', 32, 8)
_HEADS = [('continuous', 8), ('binary', 6), ('velocity', 10), ('dynamics', 16),
          ('p0_action', 400), ('p1_action', 400), ('p0_jumps', 8), ('p1_jumps', 8),
          ('p0_l_cancel', 4), ('p1_l_cancel', 4), ('p0_hurtbox', 4), ('p1_hurtbox', 4),
          ('p0_ground', 128), ('p1_ground', 128), ('p0_last_attack', 64),
          ('p1_last_attack', 64)]
_EMB_PER_FRAME = 2 * sum(d for _, _, d in _EMB) + _STAGE[2]
_FRAME = _FPF + _EMB_PER_FRAME
_NF = _K * _FPF
_NE = _K * _EMB_PER_FRAME
_IN = _FRAME * _K + _CTRL
_HOUT = sum(o for _, o in _HEADS)
_HPAD = 1280

_BT = 256


def _mlp_body(xf_ref, xe_ref, xc_ref, w1f_ref, w1e_ref, w1c_ref, b1_ref,
              w2_ref, b2_ref, wh_ref, bh_ref, out_ref):
    f32 = jnp.float32
    h = jnp.dot(xf_ref[...], w1f_ref[...], preferred_element_type=f32)
    h += jnp.dot(xe_ref[...], w1e_ref[...], preferred_element_type=f32)
    h += jnp.dot(xc_ref[...], w1c_ref[...], preferred_element_type=f32)
    h = jnp.maximum(h + b1_ref[...], 0.0).astype(jnp.bfloat16)
    h2 = jnp.dot(h, w2_ref[...], preferred_element_type=f32)
    h2 = jnp.maximum(h2 + b2_ref[...], 0.0).astype(jnp.bfloat16)
    out_ref[...] = jnp.dot(h2, wh_ref[...], preferred_element_type=f32) + bh_ref[...]


def _mlp(xf, xe, xc, w1f, w1e, w1c, b1, w2, b2, wh, bh):
    const = lambda i: (0, 0)
    return pl.pallas_call(
        _mlp_body,
        grid=(_B // _BT,),
        in_specs=[
            pl.BlockSpec((_BT, _NF), lambda i: (i, 0)),
            pl.BlockSpec((_BT, _NE), lambda i: (i, 0)),
            pl.BlockSpec((_BT, _CTRL), lambda i: (i, 0)),
            pl.BlockSpec((_NF, _HID), const),
            pl.BlockSpec((_NE, _HID), const),
            pl.BlockSpec((_CTRL, _HID), const),
            pl.BlockSpec((1, _HID), const),
            pl.BlockSpec((_HID, _TRUNK), const),
            pl.BlockSpec((1, _TRUNK), const),
            pl.BlockSpec((_TRUNK, _HPAD), const),
            pl.BlockSpec((1, _HPAD), const),
        ],
        out_specs=pl.BlockSpec((_BT, _HPAD), lambda i: (i, 0)),
        out_shape=jax.ShapeDtypeStruct((_B, _HPAD), jnp.float32),
        compiler_params=pltpu.CompilerParams(
            dimension_semantics=("arbitrary",),
        ),
    )(xf, xe, xc, w1f, w1e, w1c, b1, w2, b2, wh, bh)


_SLOT_TABLE = list(range(8)) + list(range(8)) + [8]
_TAB_DIMS = [d for _, _, d in _EMB] + [_STAGE[2]]
_TAB_VOCAB = [v for _, v, _ in _EMB] + [_STAGE[1]]
_TAB_WOFF = np.concatenate([[0], np.cumsum(
    [v * d // 2 for v, d in zip(_TAB_VOCAB, _TAB_DIMS)])]).astype(np.int32)
_TAB_WORDS = int(_TAB_WOFF[-1])
_TAB_WPAD = (_TAB_WORDS + 15) // 16 * 16
_SLOT_DW = [_TAB_DIMS[t] // 2 for t in _SLOT_TABLE]
_SLOT_CWO = np.concatenate([[0], np.cumsum(_SLOT_DW)]).astype(np.int32)
_WPF = int(_SLOT_CWO[-1])
_NSLOT = 2 * _IPP + 1
_NEW = _K * _WPF
_BC = 16
_NW = 32


def _sc_gather_body(tab_hbm, idx_hbm, out_hbm, tab_v, idx_v, out_v):
    wid = lax.axis_index("c") * 16 + lax.axis_index("s")
    pltpu.sync_copy(tab_hbm, tab_v)
    nidx = _NSLOT * _K
    rows_i = lax.iota(jnp.int32, 16) * nidx
    rows_o = lax.iota(jnp.int32, 16) * _NEW
    chunks = _B // _BC // _NW

    def chunk_body(i, carry):
        b0 = (wid * chunks + i) * _BC
        pltpu.sync_copy(idx_hbm.at[pl.ds(b0 * nidx, _BC * nidx)], idx_v)

        @plsc.parallel_loop(0, _K)
        def k_body(k):
            for s in range(_NSLOT):
                idxv = plsc.load_gather(idx_v, [rows_i + (k * _NSLOT + s)])
                wbase = idxv * _SLOT_DW[s] + int(_TAB_WOFF[_SLOT_TABLE[s]])
                obase = rows_o + (k * _WPF + int(_SLOT_CWO[s]))
                for j in range(_SLOT_DW[s]):
                    data = plsc.load_gather(tab_v, [wbase + j])
                    plsc.store_scatter(out_v, [obase + j], data)
        pltpu.sync_copy(out_v, out_hbm.at[pl.ds(b0 * _NEW, _BC * _NEW)])
        return carry

    lax.fori_loop(0, chunks, chunk_body, 0)


def _sc_gather(tab_words, idx_flat):
    mesh = plsc.VectorSubcoreMesh(core_axis_name="c", subcore_axis_name="s")
    f = functools.partial(
        pl.kernel,
        mesh=mesh,
        out_type=jax.ShapeDtypeStruct((_B * _NEW,), jnp.int32),
        scratch_types=[
            pltpu.VMEM((_TAB_WPAD,), jnp.int32),
            pltpu.VMEM((_BC * _NSLOT * _K,), jnp.int32),
            pltpu.VMEM((_BC * _NEW,), jnp.int32),
        ],
        compiler_params=pltpu.CompilerParams(needs_layout_passes=False),
    )(_sc_gather_body)
    return f(tab_words, idx_flat)


def _pack_tables(params):
    parts = []
    for name, v, d in _EMB + [_STAGE]:
        e = params[name + '_embed'].astype(jnp.bfloat16)
        parts.append(lax.bitcast_convert_type(
            e.reshape(v, d // 2, 2), jnp.int32).reshape(-1))
    flat = jnp.concatenate(parts)
    return jnp.pad(flat, (0, _TAB_WPAD - _TAB_WORDS))


def _gather_emb_sc(int_ctx, params):
    tab_words = _pack_tables(params)
    idx_flat = int_ctx.reshape(_B * _K * _NSLOT)
    words = _sc_gather(tab_words, idx_flat).reshape(_B, _NEW)
    return lax.bitcast_convert_type(words, jnp.bfloat16).reshape(_B, _NE)


def kernel(float_ctx, int_ctx, next_ctrl, params):
    bf16 = jnp.bfloat16
    xf = float_ctx.reshape(_B, _NF).astype(bf16)
    xc = next_ctrl.astype(bf16)
    xe = _gather_emb_sc(int_ctx, params)

    w1_3d = params['W1'][:_FRAME * _K].reshape(_K, _FRAME, _HID)
    w1f = w1_3d[:, :_FPF, :].reshape(_NF, _HID).astype(bf16)
    w1e = w1_3d[:, _FPF:, :].reshape(_NE, _HID).astype(bf16)
    w1c = params['W1'][_FRAME * _K:].astype(bf16)
    b1 = params['b1'].reshape(1, _HID)
    w2 = params['W2'].astype(bf16)
    b2 = params['b2'].reshape(1, _TRUNK)
    wh = jnp.concatenate([params[n + '_W'] for n, _ in _HEADS], axis=1)
    wh = jnp.pad(wh, ((0, 0), (0, _HPAD - _HOUT))).astype(bf16)
    bh = jnp.pad(jnp.concatenate([params[n + '_b'] for n, _ in _HEADS]),
                 (0, _HPAD - _HOUT)).reshape(1, _HPAD)

    out = _mlp(xf, xe, xc, w1f, w1e, w1c, b1, w2, b2, wh, bh)

    res, off = {}, 0
    for name, o in _HEADS:
        res[name] = out[:, off:off + o]
        off += o
    return res
